# Initial kernel scaffold; baseline (speedup 1.0000x reference)
import jax
import jax.numpy as jnp
from jax.experimental import pallas as pl


def kernel(x, segment_ids, cu_seqlens, beta_u, beta_z):
    raise NotImplementedError("write your pallas kernel here")


# speedup vs baseline: 139.7690x; 139.7690x over previous
import functools

import jax
import jax.numpy as jnp
from jax.experimental import pallas as pl
from jax.experimental.pallas import tpu as pltpu

_B = 16
_TB = 2048


def _mipl_body(cu_lo_ref, cu_hi_ref, x_ref, oh_ref, bu_ref, bz_ref,
               out_ref, m_ref, s_ref, n_ref, *, nblocks, tb, nseg):
    step = pl.program_id(0)

    @pl.when(step == 0)
    def _init():
        m_ref[...] = jnp.full_like(m_ref, -1e30)
        s_ref[...] = jnp.zeros_like(s_ref)
        n_ref[...] = jnp.zeros_like(n_ref)

    x = x_ref[...]
    oh = oh_ref[...]
    bu = bu_ref[...]
    bz = bz_ref[...]
    eta = bz * jax.lax.rsqrt(jnp.mean(bz * bz, axis=0, keepdims=True))
    xw = jnp.dot(x, bu, preferred_element_type=jnp.float32)
    xt = jnp.dot(x, eta, preferred_element_type=jnp.float32)

    start = step * tb
    end = start + tb
    ovl = (cu_lo_ref[...] < end) & (cu_hi_ref[...] > start)
    bmax = jnp.max(xw, axis=0, keepdims=True)
    m_old = m_ref[...]
    m_new = jnp.maximum(m_old, jnp.where(ovl, bmax, -1e30))
    scale = jnp.exp(m_old - m_new)
    m_tok = jnp.dot(oh, m_new, preferred_element_type=jnp.float32)
    e = jnp.exp(xw - m_tok)
    p = e * xt
    contract = (((0,), (0,)), ((), ()))
    s_add = jax.lax.dot_general(oh, e, contract,
                                preferred_element_type=jnp.float32)
    n_add = jax.lax.dot_general(oh, p, contract,
                                preferred_element_type=jnp.float32)
    m_ref[...] = m_new
    s_new = s_ref[...] * scale + s_add
    n_new = n_ref[...] * scale + n_add
    s_ref[...] = s_new
    n_ref[...] = n_new

    @pl.when(step == nblocks - 1)
    def _fin():
        z = jnp.where(s_new > 0, n_new / s_new, 0.0)
        bb = jnp.sqrt(jnp.mean(bz * bz, axis=0, keepdims=True))
        mean = jnp.mean(z, axis=0, keepdims=True)
        var = jnp.sum((z - mean) ** 2, axis=0, keepdims=True) / (nseg - 1)
        std = jnp.sqrt(var)
        std = jnp.where(jnp.isnan(std), 1.0, std)
        out_ref[...] = bb * (z - mean) / std


@jax.jit
def _run(x, onehot, cu_lo, cu_hi, bu2, bz2):
    t, d = x.shape
    ps = bu2.shape[1]
    nblocks = t // _TB
    body = functools.partial(_mipl_body, nblocks=nblocks, tb=_TB, nseg=_B)
    return pl.pallas_call(
        body,
        grid=(nblocks,),
        in_specs=[
            pl.BlockSpec((_B, 1), lambda i: (0, 0)),
            pl.BlockSpec((_B, 1), lambda i: (0, 0)),
            pl.BlockSpec((_TB, d), lambda i: (i, 0)),
            pl.BlockSpec((_TB, _B), lambda i: (i, 0)),
            pl.BlockSpec((d, ps), lambda i: (0, 0)),
            pl.BlockSpec((d, ps), lambda i: (0, 0)),
        ],
        out_specs=pl.BlockSpec((_B, ps), lambda i: (0, 0)),
        out_shape=jax.ShapeDtypeStruct((_B, ps), jnp.float32),
        scratch_shapes=[pltpu.VMEM((_B, ps), jnp.float32)] * 3,
        compiler_params=pltpu.CompilerParams(
            dimension_semantics=("arbitrary",)),
    )(cu_lo, cu_hi, x, onehot, bu2, bz2)


def kernel(x, segment_ids, cu_seqlens, beta_u, beta_z):
    t, d = x.shape
    p, s = beta_u.shape[1], beta_u.shape[2]
    seg = segment_ids.astype(jnp.int32)
    onehot = (seg[:, None] ==
              jnp.arange(_B, dtype=jnp.int32)[None, :]).astype(jnp.float32)
    cu = cu_seqlens.astype(jnp.int32)
    cu_lo = cu[:_B].reshape(_B, 1)
    cu_hi = cu[1:_B + 1].reshape(_B, 1)
    out = _run(x, onehot, cu_lo, cu_hi,
               beta_u.reshape(d, p * s), beta_z.reshape(d, p * s))
    return out.reshape(_B, p, s)


# speedup vs baseline: 217.8840x; 1.5589x over previous
import functools

import jax
import jax.numpy as jnp
from jax.experimental import pallas as pl
from jax.experimental.pallas import tpu as pltpu

_B = 16
_TB = 4096


def _mipl_body(cu_lo_ref, cu_hi_ref, cu_lo_row_ref, cu_hi_row_ref,
               x_ref, bu_ref, bz_ref,
               out_ref, m_ref, s_ref, n_ref, *, nblocks, tb, nseg):
    step = pl.program_id(0)

    @pl.when(step == 0)
    def _init():
        m_ref[...] = jnp.full_like(m_ref, -1e30)
        s_ref[...] = jnp.zeros_like(s_ref)
        n_ref[...] = jnp.zeros_like(n_ref)

    x = x_ref[...]
    bu = bu_ref[...]
    bz = bz_ref[...]
    eta = bz * jax.lax.rsqrt(jnp.mean(bz * bz, axis=0, keepdims=True))
    xw = jnp.dot(x, bu, preferred_element_type=jnp.float32)
    xt = jnp.dot(x, eta, preferred_element_type=jnp.float32)

    start = step * tb
    gidx = start + jax.lax.broadcasted_iota(jnp.int32, (tb, 1), 0)
    oh = ((gidx >= cu_lo_row_ref[...]) &
          (gidx < cu_hi_row_ref[...])).astype(jnp.float32)

    ovl = (cu_lo_ref[...] < start + tb) & (cu_hi_ref[...] > start)
    bmax = jnp.max(xw, axis=0, keepdims=True)
    m_old = m_ref[...]
    m_new = jnp.maximum(m_old, jnp.where(ovl, bmax, -1e30))
    scale = jnp.exp(m_old - m_new)
    m_tok = jnp.dot(oh, m_new, preferred_element_type=jnp.float32)
    e = jnp.exp(xw - m_tok)
    p = e * xt
    contract = (((0,), (0,)), ((), ()))
    s_add = jax.lax.dot_general(oh, e, contract,
                                preferred_element_type=jnp.float32)
    n_add = jax.lax.dot_general(oh, p, contract,
                                preferred_element_type=jnp.float32)
    m_ref[...] = m_new
    s_new = s_ref[...] * scale + s_add
    n_new = n_ref[...] * scale + n_add
    s_ref[...] = s_new
    n_ref[...] = n_new

    @pl.when(step == nblocks - 1)
    def _fin():
        z = jnp.where(s_new > 0, n_new / s_new, 0.0)
        bb = jnp.sqrt(jnp.mean(bz * bz, axis=0, keepdims=True))
        mean = jnp.mean(z, axis=0, keepdims=True)
        var = jnp.sum((z - mean) ** 2, axis=0, keepdims=True) / (nseg - 1)
        std = jnp.sqrt(var)
        std = jnp.where(jnp.isnan(std), 1.0, std)
        out_ref[...] = bb * (z - mean) / std


@jax.jit
def _run(x, cu_lo, cu_hi, cu_lo_row, cu_hi_row, bu2, bz2):
    t, d = x.shape
    ps = bu2.shape[1]
    nblocks = t // _TB
    body = functools.partial(_mipl_body, nblocks=nblocks, tb=_TB, nseg=_B)
    return pl.pallas_call(
        body,
        grid=(nblocks,),
        in_specs=[
            pl.BlockSpec((_B, 1), lambda i: (0, 0)),
            pl.BlockSpec((_B, 1), lambda i: (0, 0)),
            pl.BlockSpec((1, _B), lambda i: (0, 0)),
            pl.BlockSpec((1, _B), lambda i: (0, 0)),
            pl.BlockSpec((_TB, d), lambda i: (i, 0)),
            pl.BlockSpec((d, ps), lambda i: (0, 0)),
            pl.BlockSpec((d, ps), lambda i: (0, 0)),
        ],
        out_specs=pl.BlockSpec((_B, ps), lambda i: (0, 0)),
        out_shape=jax.ShapeDtypeStruct((_B, ps), jnp.float32),
        scratch_shapes=[pltpu.VMEM((_B, ps), jnp.float32)] * 3,
        compiler_params=pltpu.CompilerParams(
            dimension_semantics=("arbitrary",)),
    )(cu_lo, cu_hi, cu_lo_row, cu_hi_row, x, bu2, bz2)


def kernel(x, segment_ids, cu_seqlens, beta_u, beta_z):
    t, d = x.shape
    p, s = beta_u.shape[1], beta_u.shape[2]
    cu = cu_seqlens.astype(jnp.int32)
    cu_lo = cu[:_B].reshape(_B, 1)
    cu_hi = cu[1:_B + 1].reshape(_B, 1)
    out = _run(x, cu_lo, cu_hi, cu_lo.reshape(1, _B), cu_hi.reshape(1, _B),
               beta_u.reshape(d, p * s), beta_z.reshape(d, p * s))
    return out.reshape(_B, p, s)


# speedup vs baseline: 224.1248x; 1.0286x over previous
import functools

import jax
import jax.numpy as jnp
from jax.experimental import pallas as pl
from jax.experimental.pallas import tpu as pltpu

_B = 16
_TB = 8192


def _mipl_body(cu_lo_ref, cu_hi_ref, cu_lo_row_ref, cu_hi_row_ref,
               x_ref, bu_ref, bz_ref,
               out_ref, m_ref, s_ref, n_ref, *, nblocks, tb, nseg):
    step = pl.program_id(0)

    @pl.when(step == 0)
    def _init():
        m_ref[...] = jnp.full_like(m_ref, -1e30)
        s_ref[...] = jnp.zeros_like(s_ref)
        n_ref[...] = jnp.zeros_like(n_ref)

    x = x_ref[...]
    bu = bu_ref[...]
    bz = bz_ref[...]
    eta = bz * jax.lax.rsqrt(jnp.mean(bz * bz, axis=0, keepdims=True))
    xw = jnp.dot(x, bu, preferred_element_type=jnp.float32)
    xt = jnp.dot(x, eta, preferred_element_type=jnp.float32)

    start = step * tb
    gidx = start + jax.lax.broadcasted_iota(jnp.int32, (tb, 1), 0)
    oh = ((gidx >= cu_lo_row_ref[...]) &
          (gidx < cu_hi_row_ref[...])).astype(jnp.float32)

    ovl = (cu_lo_ref[...] < start + tb) & (cu_hi_ref[...] > start)
    bmax = jnp.max(xw, axis=0, keepdims=True)
    m_old = m_ref[...]
    m_new = jnp.maximum(m_old, jnp.where(ovl, bmax, -1e30))
    scale = jnp.exp(m_old - m_new)
    m_tok = jnp.dot(oh, m_new, preferred_element_type=jnp.float32)
    e = jnp.exp(xw - m_tok)
    p = e * xt
    contract = (((0,), (0,)), ((), ()))
    s_add = jax.lax.dot_general(oh, e, contract,
                                preferred_element_type=jnp.float32)
    n_add = jax.lax.dot_general(oh, p, contract,
                                preferred_element_type=jnp.float32)
    m_ref[...] = m_new
    s_new = s_ref[...] * scale + s_add
    n_new = n_ref[...] * scale + n_add
    s_ref[...] = s_new
    n_ref[...] = n_new

    @pl.when(step == nblocks - 1)
    def _fin():
        z = jnp.where(s_new > 0, n_new / s_new, 0.0)
        bb = jnp.sqrt(jnp.mean(bz * bz, axis=0, keepdims=True))
        mean = jnp.mean(z, axis=0, keepdims=True)
        var = jnp.sum((z - mean) ** 2, axis=0, keepdims=True) / (nseg - 1)
        std = jnp.sqrt(var)
        std = jnp.where(jnp.isnan(std), 1.0, std)
        out_ref[...] = bb * (z - mean) / std


@jax.jit
def _run(x, cu_lo, cu_hi, cu_lo_row, cu_hi_row, bu2, bz2):
    t, d = x.shape
    ps = bu2.shape[1]
    nblocks = t // _TB
    body = functools.partial(_mipl_body, nblocks=nblocks, tb=_TB, nseg=_B)
    return pl.pallas_call(
        body,
        grid=(nblocks,),
        in_specs=[
            pl.BlockSpec((_B, 1), lambda i: (0, 0)),
            pl.BlockSpec((_B, 1), lambda i: (0, 0)),
            pl.BlockSpec((1, _B), lambda i: (0, 0)),
            pl.BlockSpec((1, _B), lambda i: (0, 0)),
            pl.BlockSpec((_TB, d), lambda i: (i, 0)),
            pl.BlockSpec((d, ps), lambda i: (0, 0)),
            pl.BlockSpec((d, ps), lambda i: (0, 0)),
        ],
        out_specs=pl.BlockSpec((_B, ps), lambda i: (0, 0)),
        out_shape=jax.ShapeDtypeStruct((_B, ps), jnp.float32),
        scratch_shapes=[pltpu.VMEM((_B, ps), jnp.float32)] * 3,
        compiler_params=pltpu.CompilerParams(
            dimension_semantics=("arbitrary",)),
    )(cu_lo, cu_hi, cu_lo_row, cu_hi_row, x, bu2, bz2)


def kernel(x, segment_ids, cu_seqlens, beta_u, beta_z):
    t, d = x.shape
    p, s = beta_u.shape[1], beta_u.shape[2]
    cu = cu_seqlens.astype(jnp.int32)
    cu_lo = cu[:_B].reshape(_B, 1)
    cu_hi = cu[1:_B + 1].reshape(_B, 1)
    out = _run(x, cu_lo, cu_hi, cu_lo.reshape(1, _B), cu_hi.reshape(1, _B),
               beta_u.reshape(d, p * s), beta_z.reshape(d, p * s))
    return out.reshape(_B, p, s)
